# Initial kernel scaffold; baseline (speedup 1.0000x reference)
import jax
import jax.numpy as jnp
from jax.experimental import pallas as pl


def kernel(features, tables, W1, b1, W2, b2):
    raise NotImplementedError("write your pallas kernel here")


# speedup vs baseline: 8.1014x; 8.1014x over previous
import functools

import jax
import jax.numpy as jnp
from jax import lax
from jax.experimental import pallas as pl
from jax.experimental.pallas import tpu as pltpu
from jax.experimental.pallas import tpu_sc as plsc


def _sc_gather(table2d, idx, num_cores, num_subcores):
    n_rows, d = table2d.shape
    bf = idx.shape[0]
    nw = num_cores * num_subcores
    rows_per_w = bf // nw
    nch = 8
    ch = rows_per_w // nch

    mesh = plsc.VectorSubcoreMesh(core_axis_name="c", subcore_axis_name="s")

    @functools.partial(
        pl.kernel,
        out_type=jax.ShapeDtypeStruct((bf, d), jnp.float32),
        mesh=mesh,
        compiler_params=pltpu.CompilerParams(use_tc_tiling_on_sc=False),
        scratch_types=[
            pltpu.VMEM((rows_per_w,), jnp.int32),
            pltpu.VMEM((ch, d), jnp.float32),
            pltpu.VMEM((ch, d), jnp.float32),
            pltpu.SemaphoreType.DMA,
            pltpu.SemaphoreType.DMA,
        ],
    )
    def k(table_hbm, idx_hbm, out_hbm, idx_v, rows0, rows1, sem0, sem1):
        wid = lax.axis_index("s") * num_cores + lax.axis_index("c")
        base = wid * rows_per_w
        pltpu.sync_copy(idx_hbm.at[pl.ds(base, rows_per_w)], idx_v)
        bufs = (rows0, rows1)
        sems = (sem0, sem1)
        copies = [None] * nch
        copies[0] = pltpu.async_copy(
            table_hbm.at[idx_v.at[pl.ds(0, ch)]], bufs[0], sems[0])
        for c in range(nch):
            if c + 1 < nch:
                copies[c + 1] = pltpu.async_copy(
                    table_hbm.at[idx_v.at[pl.ds((c + 1) * ch, ch)]],
                    bufs[(c + 1) % 2], sems[(c + 1) % 2])
            copies[c].wait()
            pltpu.sync_copy(bufs[c % 2], out_hbm.at[pl.ds(base + c * ch, ch)])

    return k(table2d, idx)


def _mlp(x, w1, b1, w2row, b2, blk):
    bsz, fd = x.shape
    h = w1.shape[0]

    def body(x_ref, w1_ref, b1_ref, w2_ref, b2_ref, o_ref):
        xb = x_ref[...]
        hid = lax.dot_general(
            xb, w1_ref[...], (((1,), (1,)), ((), ())),
            preferred_element_type=jnp.float32)
        hid = jnp.maximum(hid + b1_ref[...], 0.0)
        o_ref[...] = jnp.sum(hid * w2_ref[...], axis=1) + b2_ref[0]

    return pl.pallas_call(
        body,
        grid=(bsz // blk,),
        in_specs=[
            pl.BlockSpec((blk, fd), lambda i: (i, 0)),
            pl.BlockSpec((h, fd), lambda i: (0, 0)),
            pl.BlockSpec((1, h), lambda i: (0, 0)),
            pl.BlockSpec((1, h), lambda i: (0, 0)),
            pl.BlockSpec((1,), lambda i: (0,)),
        ],
        out_specs=pl.BlockSpec((blk,), lambda i: (i,)),
        out_shape=jax.ShapeDtypeStruct((bsz,), jnp.float32),
    )(x, w1, b1.reshape(1, h), w2row.reshape(1, h), b2)


def kernel(features, tables, W1, b1, W2, b2):
    bsz, f = features.shape
    _, v, d = tables.shape
    h = W1.shape[0]
    o = W2.shape[0]

    table2d = tables.reshape(f * v, d)
    idx = (features.astype(jnp.int32)
           + (jnp.arange(f, dtype=jnp.int32) * v)[None, :]).reshape(-1)

    try:
        info = plsc.get_sparse_core_info()
        num_cores, num_subcores = info.num_cores, info.num_subcores
    except Exception:
        num_cores, num_subcores = 2, 16
    x = _sc_gather(table2d, idx, num_cores, num_subcores)
    x = x.reshape(bsz, f * d)

    out = _mlp(x, W1, b1, W2[0], b2, blk=1024)
    return out.reshape(bsz, o)
